# baseline (speedup 1.0000x reference)
import functools

import jax
import jax.numpy as jnp
from jax.experimental import pallas as pl
from jax.experimental.pallas import tpu as pltpu

_CONF_THRES = 0.25
_IOU_THRES = 0.45
_RATIO = 0.02
_MAX_DET = 300
_MAX_WH = 7680.0
_IMG_SIZE = 640.0

_N = 20000
_ROWS = 160
_LANES = 128
_NPAD = _ROWS * _LANES


def _merge(a, b):
    take = (a[0] > b[0]) | ((a[0] == b[0]) & (a[1] < b[1]))
    return [jnp.where(take, x, y) for x, y in zip(a, b)]


def _tourney(arrs):
    A = _merge([x[0:80] for x in arrs], [x[80:160] for x in arrs])
    A = _merge([x[0:40] for x in A], [x[40:80] for x in A])
    L = [x[32:40] for x in A]
    A = _merge([x[0:16] for x in A], [x[16:32] for x in A])
    A = _merge([x[0:8] for x in A], [x[8:16] for x in A])
    A = _merge(A, L)
    A = _merge([x[0:4] for x in A], [x[4:8] for x in A])
    A = _merge([x[0:2] for x in A], [x[2:4] for x in A])
    A = _merge([x[0:1] for x in A], [x[1:2] for x in A])
    w = 64
    while w >= 1:
        A = _merge([x[:, 0:w] for x in A], [x[:, w:2 * w] for x in A])
        w //= 2
    return A


def _nms_body(inp_ref, out_ref):
    f32 = jnp.float32
    i32 = jnp.int32
    x = inp_ref[...]
    xt = jnp.transpose(x)
    xt = jnp.concatenate(
        [xt, jnp.zeros((84, _NPAD - _N), f32)], axis=1)
    xt3 = xt.reshape(84, _ROWS, _LANES)
    conf = xt3[4]
    cls = jnp.zeros((_ROWS, _LANES), f32)
    for c in range(1, 80):
        s = xt3[4 + c]
        upd = s > conf
        cls = jnp.where(upd, f32(c), cls)
        conf = jnp.where(upd, s, conf)

    xc = xt3[0]
    yc = xt3[1]
    hw = xt3[2] * 0.5
    hh = xt3[3] * 0.5
    x1 = xc - hw
    y1 = yc - hh
    x2 = xc + hw
    y2 = yc + hh
    off = cls * _MAX_WH
    ox1 = x1 + off
    oy1 = y1 + off
    ox2 = x2 + off
    oy2 = y2 + off
    tarea = ((x2 - x1) * (y2 - y1)) * _IOU_THRES

    valid = conf > _CONF_THRES
    scores0 = jnp.where(valid, conf, f32(-1.0))

    flat = (jax.lax.broadcasted_iota(i32, (_ROWS, _LANES), 0) * _LANES
            + jax.lax.broadcasted_iota(i32, (_ROWS, _LANES), 1))
    lane8 = jax.lax.broadcasted_iota(i32, (8, _LANES), 1)
    row8 = jax.lax.broadcasted_iota(i32, (8, _LANES), 0)

    def body(i, state):
        scores, num, pconf, pcontrib = state
        best, fidx, bx1, by1, bx2, by2, bcl = _tourney(
            [scores, flat, x1, y1, x2, y2, cls])
        is_valid = best > 0.0
        boff = bcl * _MAX_WH
        bax = bx2 - bx1
        bay = by2 - by1
        rhs0 = _IOU_THRES * (bax * bay + 1e-7)

        ix1 = jnp.maximum(bx1 + boff, ox1)
        iy1 = jnp.maximum(by1 + boff, oy1)
        ix2 = jnp.minimum(bx2 + boff, ox2)
        iy2 = jnp.minimum(by2 + boff, oy2)
        inter = jnp.maximum(ix2 - ix1, 0.0) * jnp.maximum(iy2 - iy1, 0.0)
        sup = inter * (1.0 + _IOU_THRES) > rhs0 + tarea
        scores = jnp.where(sup | (flat == fidx), f32(-1.0), scores)

        num = num + jnp.where(is_valid, f32(1.0), f32(0.0))
        rec = is_valid & (i < 6)
        slotmask = (row8 == 0) & (lane8 == i) & rec
        pconf = jnp.where(slotmask, best, pconf)
        bcontrib = best * ((bx1 + by1 + bx2 + by2) * (1.0 / _IMG_SIZE) + bcl)
        pcontrib = jnp.where(slotmask, bcontrib, pcontrib)
        return scores, num, pconf, pcontrib

    init = (scores0, jnp.zeros((1, 1), f32), jnp.zeros((8, _LANES), f32),
            jnp.zeros((8, _LANES), f32))
    _, num, pconf, pcontrib = jax.lax.fori_loop(0, _MAX_DET, body, init)

    k = jnp.maximum(jnp.int32(1),
                    jnp.floor(num * f32(_RATIO)).astype(i32))
    usemask = (row8 == 0) & (lane8 < k)
    wsum = jnp.sum(jnp.where(usemask, pconf, 0.0), keepdims=True)
    wvsum = jnp.sum(jnp.where(usemask, pcontrib, 0.0), keepdims=True)
    target = wvsum / (2.0 * wsum)
    outv = jnp.where(num > 0.0, target, f32(0.0))
    out_ref[...] = jnp.zeros((8, _LANES), f32) + outv


@jax.jit
def kernel(model_output):
    x = model_output[0]
    out = pl.pallas_call(
        _nms_body,
        out_shape=jax.ShapeDtypeStruct((8, _LANES), jnp.float32),
    )(x)
    return out[0, 0]


# speedup vs baseline: 1.0134x; 1.0134x over previous
import functools

import jax
import jax.numpy as jnp
from jax.experimental import pallas as pl
from jax.experimental.pallas import tpu as pltpu

_CONF_THRES = 0.25
_IOU_THRES = 0.45
_RATIO = 0.02
_MAX_DET = 300
_MAX_WH = 7680.0
_IMG_SIZE = 640.0

_N = 20000
_ROWS = 160
_LANES = 128
_NPAD = _ROWS * _LANES
_BASE = 0x3E800000


def _nms_body(inp_ref, out_ref):
    f32 = jnp.float32
    i32 = jnp.int32
    u32 = jnp.uint32
    x = inp_ref[...]
    xt = jnp.transpose(x)
    xt = jnp.concatenate(
        [xt, jnp.zeros((84, _NPAD - _N), f32)], axis=1)
    xt3 = xt.reshape(84, _ROWS, _LANES)
    conf = xt3[4]
    cls = jnp.zeros((_ROWS, _LANES), f32)
    for c in range(1, 80):
        s = xt3[4 + c]
        upd = s > conf
        cls = jnp.where(upd, f32(c), cls)
        conf = jnp.where(upd, s, conf)

    xc = xt3[0]
    yc = xt3[1]
    hw = xt3[2] * 0.5
    hh = xt3[3] * 0.5
    x1 = xc - hw
    y1 = yc - hh
    x2 = xc + hw
    y2 = yc + hh
    off = cls * _MAX_WH
    ox1 = x1 + off
    oy1 = y1 + off
    ox2 = x2 + off
    oy2 = y2 + off
    tarea = ((x2 - x1) * (y2 - y1)) * _IOU_THRES

    stack = jnp.concatenate([x1, y1, x2, y2, cls], axis=1)

    rowio = jax.lax.broadcasted_iota(i32, (_ROWS, _LANES), 0)
    laneio = jax.lax.broadcasted_iota(i32, (_ROWS, _LANES), 1)
    flat = rowio * _LANES + laneio
    lane1 = jax.lax.broadcasted_iota(i32, (1, _LANES), 1)
    lane8 = jax.lax.broadcasted_iota(i32, (8, _LANES), 1)
    row8 = jax.lax.broadcasted_iota(i32, (8, _LANES), 0)

    valid = conf > _CONF_THRES
    cbits = jax.lax.bitcast_convert_type(conf, u32)
    skey0u = ((cbits - u32(_BASE)) << 8) | (u32(159) - rowio.astype(u32))
    skey0i = jax.lax.bitcast_convert_type(skey0u ^ u32(0x80000000), i32)
    skey0 = jnp.where(valid, skey0i, jnp.int32(-2**31))

    def body(i, state):
        skey, num, pconf, pcontrib = state
        t = jnp.maximum(skey[0:80], skey[80:160])
        t = jnp.maximum(t[0:40], t[40:80])
        keyA = jnp.max(t, axis=0, keepdims=True)
        keyAu = jax.lax.bitcast_convert_type(keyA, u32) ^ u32(0x80000000)
        sb = (keyAu >> 8).astype(i32)
        smax = jnp.max(sb, keepdims=True)
        is_valid = smax > 0
        roww = 159 - (keyAu & u32(255)).astype(i32)
        flatlane = roww * _LANES + lane1
        fl = jnp.where(sb == smax, flatlane, jnp.int32(2**30))
        flwin = jnp.min(fl, keepdims=True)
        best = jax.lax.bitcast_convert_type(smax + jnp.int32(_BASE), f32)

        mask = flat == flwin
        m5 = jnp.concatenate([mask] * 5, axis=1)
        g = jnp.where(m5, stack, 0.0)
        g = g[0:80] + g[80:160]
        g = g[0:40] + g[40:80]
        gs = jnp.sum(g, axis=0, keepdims=True)
        bx1 = jnp.sum(gs[:, 0:128], keepdims=True)
        by1 = jnp.sum(gs[:, 128:256], keepdims=True)
        bx2 = jnp.sum(gs[:, 256:384], keepdims=True)
        by2 = jnp.sum(gs[:, 384:512], keepdims=True)
        bcl = jnp.sum(gs[:, 512:640], keepdims=True)
        boff = bcl * _MAX_WH
        bax = bx2 - bx1
        bay = by2 - by1
        rhs0 = _IOU_THRES * (bax * bay + 1e-7)

        ix1 = jnp.maximum(bx1 + boff, ox1)
        iy1 = jnp.maximum(by1 + boff, oy1)
        ix2 = jnp.minimum(bx2 + boff, ox2)
        iy2 = jnp.minimum(by2 + boff, oy2)
        inter = jnp.maximum(ix2 - ix1, 0.0) * jnp.maximum(iy2 - iy1, 0.0)
        sup = inter * (1.0 + _IOU_THRES) > rhs0 + tarea
        skey = jnp.where(sup | mask, jnp.int32(-2**31), skey)

        num = num + jnp.where(is_valid, f32(1.0), f32(0.0))
        rec = is_valid & (i < 6)
        slotmask = (row8 == 0) & (lane8 == i) & rec
        pconf = jnp.where(slotmask, best, pconf)
        bcontrib = best * ((bx1 + by1 + bx2 + by2) * (1.0 / _IMG_SIZE) + bcl)
        pcontrib = jnp.where(slotmask, bcontrib, pcontrib)
        return skey, num, pconf, pcontrib

    init = (skey0, jnp.zeros((1, 1), f32), jnp.zeros((8, _LANES), f32),
            jnp.zeros((8, _LANES), f32))
    _, num, pconf, pcontrib = jax.lax.fori_loop(0, _MAX_DET, body, init)

    k = jnp.maximum(jnp.int32(1),
                    jnp.floor(num * f32(_RATIO)).astype(i32))
    usemask = (row8 == 0) & (lane8 < k)
    wsum = jnp.sum(jnp.where(usemask, pconf, 0.0), keepdims=True)
    wvsum = jnp.sum(jnp.where(usemask, pcontrib, 0.0), keepdims=True)
    target = wvsum / (2.0 * wsum)
    outv = jnp.where(num > 0.0, target, f32(0.0))
    out_ref[...] = jnp.zeros((8, _LANES), f32) + outv


@jax.jit
def kernel(model_output):
    x = model_output[0]
    out = pl.pallas_call(
        _nms_body,
        out_shape=jax.ShapeDtypeStruct((8, _LANES), jnp.float32),
    )(x)
    return out[0, 0]


# speedup vs baseline: 1.1114x; 1.0967x over previous
import functools

import jax
import jax.numpy as jnp
from jax.experimental import pallas as pl
from jax.experimental.pallas import tpu as pltpu

_CONF_THRES = 0.25
_IOU_THRES = 0.45
_RATIO = 0.02
_MAX_DET = 300
_MAX_WH = 7680.0
_IMG_SIZE = 640.0

_N = 20000
_ROWS = 160
_LANES = 128
_NPAD = _ROWS * _LANES


def _nms_body(inp_hbm, out_ref, xv_ref, sem):
    f32 = jnp.float32
    i32 = jnp.int32
    copy = pltpu.make_async_copy(inp_hbm, xv_ref, sem)
    copy.start()
    copy.wait()
    x = xv_ref[...]
    xt = jnp.transpose(x)
    xt = jnp.concatenate(
        [xt, jnp.zeros((84, _NPAD - _N), f32)], axis=1)
    xt3 = xt.reshape(84, _ROWS, _LANES)
    conf = xt3[4]
    cls = jnp.zeros((_ROWS, _LANES), f32)
    for c in range(1, 80):
        s = xt3[4 + c]
        upd = s > conf
        cls = jnp.where(upd, f32(c), cls)
        conf = jnp.where(upd, s, conf)

    xc = xt3[0]
    yc = xt3[1]
    hw = xt3[2] * 0.5
    hh = xt3[3] * 0.5
    x1 = xc - hw
    y1 = yc - hh
    x2 = xc + hw
    y2 = yc + hh
    off = cls * _MAX_WH
    ox1 = x1 + off
    oy1 = y1 + off
    ox2 = x2 + off
    oy2 = y2 + off
    tarea = ((x2 - x1) * (y2 - y1)) * _IOU_THRES

    valid = conf > _CONF_THRES
    scores0 = jnp.where(valid, conf, f32(-1.0))

    stack = jnp.concatenate([x1, y1, x2, y2, cls], axis=1)

    flat = (jax.lax.broadcasted_iota(i32, (_ROWS, _LANES), 0) * _LANES
            + jax.lax.broadcasted_iota(i32, (_ROWS, _LANES), 1))
    lane8 = jax.lax.broadcasted_iota(i32, (8, _LANES), 1)
    row8 = jax.lax.broadcasted_iota(i32, (8, _LANES), 0)

    def tmax(v):
        v = jnp.maximum(v[0:80], v[80:160])
        v = jnp.maximum(v[0:40], v[40:80])
        return jnp.max(v, keepdims=True)

    def tmin(v):
        v = jnp.minimum(v[0:80], v[80:160])
        v = jnp.minimum(v[0:40], v[40:80])
        return jnp.min(v, keepdims=True)

    def body(i, state):
        scores, num, pconf, pcontrib = state
        best = tmax(scores)
        is_valid = best > 0.0
        m1 = scores >= best
        fidx = tmin(jnp.where(m1, flat, jnp.int32(2**30)))
        mask = flat == fidx
        m5 = jnp.concatenate([mask] * 5, axis=1)
        g = jnp.where(m5, stack, 0.0)
        g = g[0:80] + g[80:160]
        g = g[0:40] + g[40:80]
        gs = jnp.sum(g, axis=0, keepdims=True)
        bx1 = jnp.sum(gs[:, 0:128], keepdims=True)
        by1 = jnp.sum(gs[:, 128:256], keepdims=True)
        bx2 = jnp.sum(gs[:, 256:384], keepdims=True)
        by2 = jnp.sum(gs[:, 384:512], keepdims=True)
        bcl = jnp.sum(gs[:, 512:640], keepdims=True)
        boff = bcl * _MAX_WH
        bax = bx2 - bx1
        bay = by2 - by1
        rhs0 = _IOU_THRES * (bax * bay + 1e-7)

        ix1 = jnp.maximum(bx1 + boff, ox1)
        iy1 = jnp.maximum(by1 + boff, oy1)
        ix2 = jnp.minimum(bx2 + boff, ox2)
        iy2 = jnp.minimum(by2 + boff, oy2)
        inter = jnp.maximum(ix2 - ix1, 0.0) * jnp.maximum(iy2 - iy1, 0.0)
        sup = inter * (1.0 + _IOU_THRES) > rhs0 + tarea
        scores = jnp.where(sup | mask, f32(-1.0), scores)

        num = num + jnp.where(is_valid, f32(1.0), f32(0.0))
        rec = is_valid & (i < 6)
        slotmask = (row8 == 0) & (lane8 == i) & rec
        pconf = jnp.where(slotmask, best, pconf)
        bcontrib = best * ((bx1 + by1 + bx2 + by2) * (1.0 / _IMG_SIZE) + bcl)
        pcontrib = jnp.where(slotmask, bcontrib, pcontrib)
        return scores, num, pconf, pcontrib

    init = (scores0, jnp.zeros((1, 1), f32), jnp.zeros((8, _LANES), f32),
            jnp.zeros((8, _LANES), f32))
    _, num, pconf, pcontrib = jax.lax.fori_loop(0, _MAX_DET, body, init)

    k = jnp.maximum(jnp.int32(1),
                    jnp.floor(num * f32(_RATIO)).astype(i32))
    usemask = (row8 == 0) & (lane8 < k)
    wsum = jnp.sum(jnp.where(usemask, pconf, 0.0), keepdims=True)
    wvsum = jnp.sum(jnp.where(usemask, pcontrib, 0.0), keepdims=True)
    target = wvsum / (2.0 * wsum)
    outv = jnp.where(num > 0.0, target, f32(0.0))
    out_ref[...] = jnp.zeros((8, _LANES), f32) + outv


@jax.jit
def kernel(model_output):
    x = model_output[0]
    out = pl.pallas_call(
        _nms_body,
        in_specs=[pl.BlockSpec(memory_space=pl.ANY)],
        out_shape=jax.ShapeDtypeStruct((8, _LANES), jnp.float32),
        scratch_shapes=[pltpu.VMEM((_N, 84), jnp.float32),
                        pltpu.SemaphoreType.DMA],
    )(x)
    return out[0, 0]


# speedup vs baseline: 1.2583x; 1.1322x over previous
import functools

import jax
import jax.numpy as jnp
from jax.experimental import pallas as pl
from jax.experimental.pallas import tpu as pltpu

_CONF_THRES = 0.25
_IOU_THRES = 0.45
_RATIO = 0.02
_MAX_DET = 300
_MAX_WH = 7680.0
_IMG_SIZE = 640.0

_N = 20000
_ROWS = 160
_LANES = 128
_NPAD = _ROWS * _LANES


def _nms_body(inp_hbm, out_ref, xv_ref, sem):
    f32 = jnp.float32
    i32 = jnp.int32
    copy = pltpu.make_async_copy(inp_hbm.at[0], xv_ref, sem)
    copy.start()
    copy.wait()
    x = xv_ref[...]
    xt = jnp.transpose(x)
    xt = jnp.concatenate(
        [xt, jnp.zeros((84, _NPAD - _N), f32)], axis=1)
    xt3 = xt.reshape(84, _ROWS, _LANES)
    conf = xt3[4]
    cls = jnp.zeros((_ROWS, _LANES), f32)
    for c in range(1, 80):
        s = xt3[4 + c]
        upd = s > conf
        cls = jnp.where(upd, f32(c), cls)
        conf = jnp.where(upd, s, conf)

    xc = xt3[0]
    yc = xt3[1]
    hw = xt3[2] * 0.5
    hh = xt3[3] * 0.5
    x1 = xc - hw
    y1 = yc - hh
    x2 = xc + hw
    y2 = yc + hh
    off = cls * _MAX_WH
    ox1 = x1 + off
    oy1 = y1 + off
    ox2 = x2 + off
    oy2 = y2 + off
    tarea = ((x2 - x1) * (y2 - y1)) * _IOU_THRES

    valid = conf > _CONF_THRES
    scores0 = jnp.where(valid, conf, f32(-1.0))

    stack = jnp.concatenate([x1, y1, x2, y2, cls], axis=1)

    flat = (jax.lax.broadcasted_iota(i32, (_ROWS, _LANES), 0) * _LANES
            + jax.lax.broadcasted_iota(i32, (_ROWS, _LANES), 1))
    lane8 = jax.lax.broadcasted_iota(i32, (8, _LANES), 1)
    row8 = jax.lax.broadcasted_iota(i32, (8, _LANES), 0)

    def tmax(v):
        v = jnp.maximum(v[0:80], v[80:160])
        v = jnp.maximum(v[0:40], v[40:80])
        return jnp.max(v, keepdims=True)

    def tmin(v):
        v = jnp.minimum(v[0:80], v[80:160])
        v = jnp.minimum(v[0:40], v[40:80])
        return jnp.min(v, keepdims=True)

    def body(i, state):
        scores, num, pconf, pcontrib = state
        best = tmax(scores)
        is_valid = best > 0.0
        m1 = scores >= best
        fidx = tmin(jnp.where(m1, flat, jnp.int32(2**30)))
        mask = flat == fidx
        m5 = jnp.concatenate([mask] * 5, axis=1)
        g = jnp.where(m5, stack, 0.0)
        g = g[0:80] + g[80:160]
        g = g[0:40] + g[40:80]
        gs = jnp.sum(g, axis=0, keepdims=True)
        bx1 = jnp.sum(gs[:, 0:128], keepdims=True)
        by1 = jnp.sum(gs[:, 128:256], keepdims=True)
        bx2 = jnp.sum(gs[:, 256:384], keepdims=True)
        by2 = jnp.sum(gs[:, 384:512], keepdims=True)
        bcl = jnp.sum(gs[:, 512:640], keepdims=True)
        boff = bcl * _MAX_WH
        bax = bx2 - bx1
        bay = by2 - by1
        rhs0 = _IOU_THRES * (bax * bay + 1e-7)

        ix1 = jnp.maximum(bx1 + boff, ox1)
        iy1 = jnp.maximum(by1 + boff, oy1)
        ix2 = jnp.minimum(bx2 + boff, ox2)
        iy2 = jnp.minimum(by2 + boff, oy2)
        inter = jnp.maximum(ix2 - ix1, 0.0) * jnp.maximum(iy2 - iy1, 0.0)
        sup = inter * (1.0 + _IOU_THRES) > rhs0 + tarea
        scores = jnp.where(sup | mask, f32(-1.0), scores)

        num = num + jnp.where(is_valid, f32(1.0), f32(0.0))
        rec = is_valid & (i < 6)
        slotmask = (row8 == 0) & (lane8 == i) & rec
        pconf = jnp.where(slotmask, best, pconf)
        bcontrib = best * ((bx1 + by1 + bx2 + by2) * (1.0 / _IMG_SIZE) + bcl)
        pcontrib = jnp.where(slotmask, bcontrib, pcontrib)
        return scores, num, pconf, pcontrib

    init = (scores0, jnp.zeros((1, 1), f32), jnp.zeros((8, _LANES), f32),
            jnp.zeros((8, _LANES), f32))
    _, num, pconf, pcontrib = jax.lax.fori_loop(0, _MAX_DET, body, init)

    k = jnp.maximum(jnp.int32(1),
                    jnp.floor(num * f32(_RATIO)).astype(i32))
    usemask = (row8 == 0) & (lane8 < k)
    wsum = jnp.sum(jnp.where(usemask, pconf, 0.0), keepdims=True)
    wvsum = jnp.sum(jnp.where(usemask, pcontrib, 0.0), keepdims=True)
    target = wvsum / (2.0 * wsum)
    outv = jnp.where(num > 0.0, target, f32(0.0))
    out_ref[...] = jnp.zeros((8, _LANES), f32) + outv


@jax.jit
def kernel(model_output):
    out = pl.pallas_call(
        _nms_body,
        in_specs=[pl.BlockSpec(memory_space=pl.ANY)],
        out_shape=jax.ShapeDtypeStruct((8, _LANES), jnp.float32),
        scratch_shapes=[pltpu.VMEM((_N, 84), jnp.float32),
                        pltpu.SemaphoreType.DMA],
    )(model_output)
    return out[0, 0]
